# baseline (speedup 1.0000x reference)
import functools

import jax
import jax.numpy as jnp
import numpy as np
from jax import lax
from jax.experimental import pallas as pl
from jax.experimental.pallas import tpu as pltpu
from jax.experimental.pallas import tpu_sc as plsc

_N_ATOMS = 10000
_M_NBRS = 32
_NDIM = 3
_NBR_CUTOFF = 3.0
_START = 0.0
_STOP = 12.0
_N_CELLS = 64
_N_NBR_CELLS = 27
_N_CAND = _N_NBR_CELLS * _M_NBRS
_PAD_CAND = 896

_BLOCK_B = 1000


def _cell_grid():
    r = jnp.arange(_START, _STOP, _NBR_CUTOFF, dtype=jnp.float32)
    mesh = jnp.meshgrid(*([r] * _NDIM))
    return jnp.transpose(jnp.stack(mesh)).reshape(_N_CELLS, _NDIM)


def _closest_atoms_kernel(cells_ref, coordst_ref, out_ref):
    c0 = cells_ref[:, 0:1]
    c1 = cells_ref[:, 1:2]
    c2 = cells_ref[:, 2:3]
    x0 = coordst_ref[0:1, :]
    x1 = coordst_ref[1:2, :]
    x2 = coordst_ref[2:3, :]
    d = ((c0 - x0) ** 2 + (c1 - x1) ** 2) + (c2 - x2) ** 2
    iota = jax.lax.broadcasted_iota(jnp.int32, (_N_CELLS, _N_ATOMS), 1)
    big = jnp.int32(1 << 30)
    for k in range(_M_NBRS):
        vmin = jnp.min(d, axis=1, keepdims=True)
        idx = jnp.min(jnp.where(d == vmin, iota, big), axis=1, keepdims=True)
        out_ref[:, k : k + 1] = idx.astype(jnp.float32)
        d = jnp.where(iota == idx, jnp.float32(1e30), d)


def _make_sc_gather(v_rows, d_lanes, n_idx):
    info = plsc.get_sparse_core_info()
    nw = info.num_cores * info.num_subcores
    b_per_w = n_idx // nw
    mesh = plsc.VectorSubcoreMesh(core_axis_name="c", subcore_axis_name="s")

    @functools.partial(
        pl.kernel,
        mesh=mesh,
        compiler_params=pltpu.CompilerParams(use_tc_tiling_on_sc=False),
        out_type=jax.ShapeDtypeStruct((n_idx, d_lanes), jnp.float32),
        scratch_types=[
            pltpu.VMEM((b_per_w,), jnp.int32),
            pltpu.VMEM((b_per_w, d_lanes), jnp.float32),
            pltpu.SemaphoreType.DMA,
        ],
    )
    def gather_k(table_hbm, idx_hbm, out_hbm, idx_v, rows_v, sem):
        wid = lax.axis_index("s") * info.num_cores + lax.axis_index("c")
        base = wid * b_per_w
        pltpu.sync_copy(idx_hbm.at[pl.ds(base, b_per_w)], idx_v)
        pltpu.async_copy(table_hbm.at[idx_v], rows_v, sem).wait()
        pltpu.sync_copy(rows_v, out_hbm.at[pl.ds(base, b_per_w)])

    return gather_k


def _nbr_list_kernel(cellst_ref, coords_ref, tab_ref, out_ref):
    b = pl.program_id(0)
    x0 = coords_ref[:, 0:1]
    x1 = coords_ref[:, 1:2]
    x2 = coords_ref[:, 2:3]
    c0 = cellst_ref[0:1, :]
    c1 = cellst_ref[1:2, :]
    c2 = cellst_ref[2:3, :]
    dca = ((c0 - x0) ** 2 + (c1 - x1) ** 2) + (c2 - x2) ** 2
    iota_c = jax.lax.broadcasted_iota(jnp.int32, (_BLOCK_B, _N_CELLS), 1)
    vmin = jnp.min(dca, axis=1, keepdims=True)
    cid = jnp.min(
        jnp.where(dca == vmin, iota_c, jnp.int32(99)), axis=1, keepdims=True
    )
    onehot = (iota_c == cid).astype(jnp.float32)

    g = jnp.dot(
        onehot,
        tab_ref[...],
        preferred_element_type=jnp.float32,
        precision=jax.lax.Precision.HIGHEST,
    )
    cc0 = g[:, 0 * _PAD_CAND : 0 * _PAD_CAND + _PAD_CAND]
    cc1 = g[:, 1 * _PAD_CAND : 1 * _PAD_CAND + _PAD_CAND]
    cc2 = g[:, 2 * _PAD_CAND : 2 * _PAD_CAND + _PAD_CAND]
    ids = g[:, 3 * _PAD_CAND : 3 * _PAD_CAND + _PAD_CAND]
    dup = g[:, 4 * _PAD_CAND : 4 * _PAD_CAND + _PAD_CAND]

    d = ((x0 - cc0) ** 2 + (x1 - cc1) ** 2) + (x2 - cc2) ** 2
    rowid = (
        (b * _BLOCK_B)
        + jax.lax.broadcasted_iota(jnp.int32, (_BLOCK_B, 1), 0)
    ).astype(jnp.float32)
    d = jnp.where((dup > 0.5) | (ids == rowid), jnp.float32(1e20), d)

    iota_j = jax.lax.broadcasted_iota(jnp.int32, (_BLOCK_B, _PAD_CAND), 1)
    nchunk = _PAD_CAND // 128
    ids3 = ids.reshape(_BLOCK_B, nchunk, 128)
    iota_c7 = jax.lax.broadcasted_iota(jnp.int32, (_BLOCK_B, nchunk), 1)
    big = jnp.int32(1 << 30)
    for k in range(_M_NBRS):
        vmin = jnp.min(d, axis=1, keepdims=True)
        sel = jnp.where(d == vmin, iota_j, big)
        jmin = jnp.min(sel, axis=1, keepdims=True)
        chunk = jax.lax.div(jmin, 128)
        lane = jax.lax.rem(jmin, 128)
        lane3 = jnp.broadcast_to(lane[:, None, :], (_BLOCK_B, nchunk, 1))
        val7 = jnp.take_along_axis(ids3, lane3, axis=2)[:, :, 0]
        out_ref[:, k : k + 1] = jnp.sum(
            jnp.where(iota_c7 == chunk, val7, 0.0), axis=1, keepdims=True
        )
        d = jnp.where(sel == jmin, jnp.float32(1e30), d)


@jax.jit
def kernel(coords):
    coords = coords.astype(jnp.float32)
    cells = _cell_grid()
    cellst = cells.T

    d_cc = jnp.sum((cells[:, None, :] - cells[None, :, :]) ** 2, axis=-1)
    _, nbr_cells = jax.lax.top_k(-d_cc, _N_NBR_CELLS)

    closest_f = pl.pallas_call(
        _closest_atoms_kernel,
        out_shape=jax.ShapeDtypeStruct((_N_CELLS, _M_NBRS), jnp.float32),
    )(cells, coords.T)
    closest_atoms = closest_f.astype(jnp.int32)

    cand = closest_atoms[nbr_cells].reshape(_N_CELLS, _N_CAND)
    order = jnp.argsort(cand, axis=1)
    sorted_c = jnp.take_along_axis(cand, order, axis=1)
    dup_sorted = jnp.concatenate(
        [
            jnp.zeros((_N_CELLS, 1), dtype=bool),
            sorted_c[:, 1:] == sorted_c[:, :-1],
        ],
        axis=1,
    )
    inv = jnp.argsort(order, axis=1)
    dup = jnp.take_along_axis(dup_sorted, inv, axis=1)

    coords_pad = jnp.pad(coords, ((0, 0), (0, 16 - _NDIM)))
    sc_gather = _make_sc_gather(_N_ATOMS, 16, _N_CELLS * _N_CAND)
    gathered = sc_gather(coords_pad, cand.reshape(-1))
    cand_coords = gathered[:, :_NDIM].reshape(_N_CELLS, _N_CAND, _NDIM)

    npad = _PAD_CAND - _N_CAND
    far = jnp.full((_N_CELLS, npad), 1e15, jnp.float32)
    zpad = jnp.zeros((_N_CELLS, npad), jnp.float32)
    tab = jnp.concatenate(
        [
            cand_coords[:, :, 0], far,
            cand_coords[:, :, 1], far,
            cand_coords[:, :, 2], far,
            cand.astype(jnp.float32), zpad,
            dup.astype(jnp.float32), zpad,
        ],
        axis=1,
    )

    nblocks = _N_ATOMS // _BLOCK_B
    out_f = pl.pallas_call(
        _nbr_list_kernel,
        grid=(nblocks,),
        in_specs=[
            pl.BlockSpec((_NDIM, _N_CELLS), lambda i: (0, 0)),
            pl.BlockSpec((_BLOCK_B, _NDIM), lambda i: (i, 0)),
            pl.BlockSpec((_N_CELLS, 5 * _PAD_CAND), lambda i: (0, 0)),
        ],
        out_specs=pl.BlockSpec((_BLOCK_B, _M_NBRS), lambda i: (i, 0)),
        out_shape=jax.ShapeDtypeStruct((_N_ATOMS, _M_NBRS), jnp.float32),
        compiler_params=pltpu.CompilerParams(
            dimension_semantics=("parallel",)
        ),
    )(cellst, coords, tab)
    return out_f.astype(jnp.int32)


# speedup vs baseline: 2.3914x; 2.3914x over previous
import functools

import jax
import jax.numpy as jnp
import numpy as np
from jax import lax
from jax.experimental import pallas as pl
from jax.experimental.pallas import tpu as pltpu
from jax.experimental.pallas import tpu_sc as plsc

_N_ATOMS = 10000
_M_NBRS = 32
_NDIM = 3
_NBR_CUTOFF = 3.0
_START = 0.0
_STOP = 12.0
_N_CELLS = 64
_N_NBR_CELLS = 27
_N_CAND = _N_NBR_CELLS * _M_NBRS
_PAD_CAND = 896

_BLOCK_B = 1000


def _cell_grid():
    r = jnp.arange(_START, _STOP, _NBR_CUTOFF, dtype=jnp.float32)
    mesh = jnp.meshgrid(*([r] * _NDIM))
    return jnp.transpose(jnp.stack(mesh)).reshape(_N_CELLS, _NDIM)


def _closest_atoms_kernel(cells_ref, coordst_ref, out_ref):
    c0 = cells_ref[:, 0:1]
    c1 = cells_ref[:, 1:2]
    c2 = cells_ref[:, 2:3]
    x0 = coordst_ref[0:1, :]
    x1 = coordst_ref[1:2, :]
    x2 = coordst_ref[2:3, :]
    d = ((c0 - x0) ** 2 + (c1 - x1) ** 2) + (c2 - x2) ** 2
    iota = jax.lax.broadcasted_iota(jnp.int32, (_N_CELLS, _N_ATOMS), 1)
    big = jnp.int32(1 << 30)
    for k in range(_M_NBRS):
        vmin = jnp.min(d, axis=1, keepdims=True)
        idx = jnp.min(jnp.where(d == vmin, iota, big), axis=1, keepdims=True)
        out_ref[:, k : k + 1] = idx.astype(jnp.float32)
        d = jnp.where(iota == idx, jnp.float32(1e30), d)


def _make_sc_gather(v_rows, d_lanes, n_idx):
    info = plsc.get_sparse_core_info()
    nw = info.num_cores * info.num_subcores
    b_per_w = n_idx // nw
    mesh = plsc.VectorSubcoreMesh(core_axis_name="c", subcore_axis_name="s")

    @functools.partial(
        pl.kernel,
        mesh=mesh,
        compiler_params=pltpu.CompilerParams(use_tc_tiling_on_sc=False),
        out_type=jax.ShapeDtypeStruct((n_idx, d_lanes), jnp.float32),
        scratch_types=[
            pltpu.VMEM((b_per_w,), jnp.int32),
            pltpu.VMEM((b_per_w, d_lanes), jnp.float32),
            pltpu.SemaphoreType.DMA,
        ],
    )
    def gather_k(table_hbm, idx_hbm, out_hbm, idx_v, rows_v, sem):
        wid = lax.axis_index("s") * info.num_cores + lax.axis_index("c")
        base = wid * b_per_w
        pltpu.sync_copy(idx_hbm.at[pl.ds(base, b_per_w)], idx_v)
        pltpu.async_copy(table_hbm.at[idx_v], rows_v, sem).wait()
        pltpu.sync_copy(rows_v, out_hbm.at[pl.ds(base, b_per_w)])

    return gather_k


def _nbr_list_kernel(cellst_ref, coords_ref, tab_ref, out_ref):
    b = pl.program_id(0)
    x0 = coords_ref[:, 0:1]
    x1 = coords_ref[:, 1:2]
    x2 = coords_ref[:, 2:3]
    c0 = cellst_ref[0:1, :]
    c1 = cellst_ref[1:2, :]
    c2 = cellst_ref[2:3, :]
    dca = ((c0 - x0) ** 2 + (c1 - x1) ** 2) + (c2 - x2) ** 2
    iota_c = jax.lax.broadcasted_iota(jnp.int32, (_BLOCK_B, _N_CELLS), 1)
    vmin = jnp.min(dca, axis=1, keepdims=True)
    cid = jnp.min(
        jnp.where(dca == vmin, iota_c, jnp.int32(99)), axis=1, keepdims=True
    )
    onehot = (iota_c == cid).astype(jnp.float32)

    g = jnp.dot(
        onehot,
        tab_ref[...],
        preferred_element_type=jnp.float32,
        precision=jax.lax.Precision.HIGHEST,
    )
    cc0 = g[:, 0 * _PAD_CAND : 0 * _PAD_CAND + _PAD_CAND]
    cc1 = g[:, 1 * _PAD_CAND : 1 * _PAD_CAND + _PAD_CAND]
    cc2 = g[:, 2 * _PAD_CAND : 2 * _PAD_CAND + _PAD_CAND]
    ids = g[:, 3 * _PAD_CAND : 3 * _PAD_CAND + _PAD_CAND]
    dup = g[:, 4 * _PAD_CAND : 4 * _PAD_CAND + _PAD_CAND]

    d = ((x0 - cc0) ** 2 + (x1 - cc1) ** 2) + (x2 - cc2) ** 2
    rowid = (
        (b * _BLOCK_B)
        + jax.lax.broadcasted_iota(jnp.int32, (_BLOCK_B, 1), 0)
    ).astype(jnp.float32)
    d = jnp.where((dup > 0.5) | (ids == rowid), jnp.float32(1e20), d)

    iota_j = jax.lax.broadcasted_iota(jnp.int32, (_BLOCK_B, _PAD_CAND), 1)
    big = jnp.int32(1 << 30)
    for k in range(_M_NBRS):
        vmin = jnp.min(d, axis=1, keepdims=True)
        sel = jnp.where(d == vmin, iota_j, big)
        jmin = jnp.min(sel, axis=1, keepdims=True)
        hit = sel == jmin
        out_ref[:, k : k + 1] = jnp.sum(
            jnp.where(hit, ids, 0.0), axis=1, keepdims=True
        )
        d = jnp.where(hit, jnp.float32(1e30), d)


@jax.jit
def kernel(coords):
    coords = coords.astype(jnp.float32)
    cells = _cell_grid()
    cellst = cells.T

    d_cc = jnp.sum((cells[:, None, :] - cells[None, :, :]) ** 2, axis=-1)
    _, nbr_cells = jax.lax.top_k(-d_cc, _N_NBR_CELLS)

    closest_f = pl.pallas_call(
        _closest_atoms_kernel,
        out_shape=jax.ShapeDtypeStruct((_N_CELLS, _M_NBRS), jnp.float32),
    )(cells, coords.T)
    closest_atoms = closest_f.astype(jnp.int32)

    cand = closest_atoms[nbr_cells].reshape(_N_CELLS, _N_CAND)
    order = jnp.argsort(cand, axis=1)
    sorted_c = jnp.take_along_axis(cand, order, axis=1)
    dup_sorted = jnp.concatenate(
        [
            jnp.zeros((_N_CELLS, 1), dtype=bool),
            sorted_c[:, 1:] == sorted_c[:, :-1],
        ],
        axis=1,
    )
    inv = jnp.argsort(order, axis=1)
    dup = jnp.take_along_axis(dup_sorted, inv, axis=1)

    coords_pad = jnp.pad(coords, ((0, 0), (0, 16 - _NDIM)))
    sc_gather = _make_sc_gather(_N_ATOMS, 16, _N_CELLS * _N_CAND)
    gathered = sc_gather(coords_pad, cand.reshape(-1))
    cand_coords = gathered[:, :_NDIM].reshape(_N_CELLS, _N_CAND, _NDIM)

    npad = _PAD_CAND - _N_CAND
    far = jnp.full((_N_CELLS, npad), 1e15, jnp.float32)
    zpad = jnp.zeros((_N_CELLS, npad), jnp.float32)
    tab = jnp.concatenate(
        [
            cand_coords[:, :, 0], far,
            cand_coords[:, :, 1], far,
            cand_coords[:, :, 2], far,
            cand.astype(jnp.float32), zpad,
            dup.astype(jnp.float32), zpad,
        ],
        axis=1,
    )

    nblocks = _N_ATOMS // _BLOCK_B
    out_f = pl.pallas_call(
        _nbr_list_kernel,
        grid=(nblocks,),
        in_specs=[
            pl.BlockSpec((_NDIM, _N_CELLS), lambda i: (0, 0)),
            pl.BlockSpec((_BLOCK_B, _NDIM), lambda i: (i, 0)),
            pl.BlockSpec((_N_CELLS, 5 * _PAD_CAND), lambda i: (0, 0)),
        ],
        out_specs=pl.BlockSpec((_BLOCK_B, _M_NBRS), lambda i: (i, 0)),
        out_shape=jax.ShapeDtypeStruct((_N_ATOMS, _M_NBRS), jnp.float32),
        compiler_params=pltpu.CompilerParams(
            dimension_semantics=("parallel",)
        ),
    )(cellst, coords, tab)
    return out_f.astype(jnp.int32)


# speedup vs baseline: 2.6121x; 1.0923x over previous
import functools

import jax
import jax.numpy as jnp
import numpy as np
from jax import lax
from jax.experimental import pallas as pl
from jax.experimental.pallas import tpu as pltpu
from jax.experimental.pallas import tpu_sc as plsc

_N_ATOMS = 10000
_M_NBRS = 32
_NDIM = 3
_NBR_CUTOFF = 3.0
_START = 0.0
_STOP = 12.0
_N_CELLS = 64
_N_NBR_CELLS = 27
_N_CAND = _N_NBR_CELLS * _M_NBRS
_PAD_CAND = 896

_BLOCK_B = 2000


def _cell_grid():
    r = jnp.arange(_START, _STOP, _NBR_CUTOFF, dtype=jnp.float32)
    mesh = jnp.meshgrid(*([r] * _NDIM))
    return jnp.transpose(jnp.stack(mesh)).reshape(_N_CELLS, _NDIM)


def _closest_atoms_kernel(cells_ref, coordst_ref, out_ref):
    c0 = cells_ref[:, 0:1]
    c1 = cells_ref[:, 1:2]
    c2 = cells_ref[:, 2:3]
    x0 = coordst_ref[0:1, :]
    x1 = coordst_ref[1:2, :]
    x2 = coordst_ref[2:3, :]
    d = ((c0 - x0) ** 2 + (c1 - x1) ** 2) + (c2 - x2) ** 2
    iota = jax.lax.broadcasted_iota(jnp.int32, (_N_CELLS, _N_ATOMS), 1)
    big = jnp.int32(1 << 30)
    for k in range(_M_NBRS):
        vmin = jnp.min(d, axis=1, keepdims=True)
        idx = jnp.min(jnp.where(d == vmin, iota, big), axis=1, keepdims=True)
        out_ref[:, k : k + 1] = idx.astype(jnp.float32)
        d = jnp.where(iota == idx, jnp.float32(1e30), d)


def _make_sc_gather(v_rows, d_lanes, n_idx):
    info = plsc.get_sparse_core_info()
    nw = info.num_cores * info.num_subcores
    b_per_w = n_idx // nw
    mesh = plsc.VectorSubcoreMesh(core_axis_name="c", subcore_axis_name="s")

    @functools.partial(
        pl.kernel,
        mesh=mesh,
        compiler_params=pltpu.CompilerParams(use_tc_tiling_on_sc=False),
        out_type=jax.ShapeDtypeStruct((n_idx, d_lanes), jnp.float32),
        scratch_types=[
            pltpu.VMEM((b_per_w,), jnp.int32),
            pltpu.VMEM((b_per_w, d_lanes), jnp.float32),
            pltpu.SemaphoreType.DMA,
        ],
    )
    def gather_k(table_hbm, idx_hbm, out_hbm, idx_v, rows_v, sem):
        wid = lax.axis_index("s") * info.num_cores + lax.axis_index("c")
        base = wid * b_per_w
        pltpu.sync_copy(idx_hbm.at[pl.ds(base, b_per_w)], idx_v)
        pltpu.async_copy(table_hbm.at[idx_v], rows_v, sem).wait()
        pltpu.sync_copy(rows_v, out_hbm.at[pl.ds(base, b_per_w)])

    return gather_k


def _nbr_list_kernel(cellst_ref, coords_ref, tab_ref, out_ref):
    b = pl.program_id(0)
    x0 = coords_ref[:, 0:1]
    x1 = coords_ref[:, 1:2]
    x2 = coords_ref[:, 2:3]
    c0 = cellst_ref[0:1, :]
    c1 = cellst_ref[1:2, :]
    c2 = cellst_ref[2:3, :]
    dca = ((c0 - x0) ** 2 + (c1 - x1) ** 2) + (c2 - x2) ** 2
    iota_c = jax.lax.broadcasted_iota(jnp.int32, (_BLOCK_B, _N_CELLS), 1)
    vmin = jnp.min(dca, axis=1, keepdims=True)
    cid = jnp.min(
        jnp.where(dca == vmin, iota_c, jnp.int32(99)), axis=1, keepdims=True
    )
    onehot = (iota_c == cid).astype(jnp.float32)

    g = jnp.dot(
        onehot,
        tab_ref[...],
        preferred_element_type=jnp.float32,
        precision=jax.lax.Precision.HIGHEST,
    )
    cc0 = g[:, 0 * _PAD_CAND : 0 * _PAD_CAND + _PAD_CAND]
    cc1 = g[:, 1 * _PAD_CAND : 1 * _PAD_CAND + _PAD_CAND]
    cc2 = g[:, 2 * _PAD_CAND : 2 * _PAD_CAND + _PAD_CAND]
    ids = g[:, 3 * _PAD_CAND : 3 * _PAD_CAND + _PAD_CAND]
    dup = g[:, 4 * _PAD_CAND : 4 * _PAD_CAND + _PAD_CAND]

    d = ((x0 - cc0) ** 2 + (x1 - cc1) ** 2) + (x2 - cc2) ** 2
    rowid = (
        (b * _BLOCK_B)
        + jax.lax.broadcasted_iota(jnp.int32, (_BLOCK_B, 1), 0)
    ).astype(jnp.float32)
    d = jnp.where((dup > 0.5) | (ids == rowid), jnp.float32(1e20), d)

    iota_j = jax.lax.broadcasted_iota(jnp.int32, (_BLOCK_B, _PAD_CAND), 1)
    big = jnp.int32(1 << 30)
    for k in range(_M_NBRS):
        vmin = jnp.min(d, axis=1, keepdims=True)
        sel = jnp.where(d == vmin, iota_j, big)
        jmin = jnp.min(sel, axis=1, keepdims=True)
        hit = sel == jmin
        out_ref[:, k : k + 1] = jnp.sum(
            jnp.where(hit, ids, 0.0), axis=1, keepdims=True
        )
        d = jnp.where(hit, jnp.float32(1e30), d)


@jax.jit
def kernel(coords):
    coords = coords.astype(jnp.float32)
    cells = _cell_grid()
    cellst = cells.T

    d_cc = jnp.sum((cells[:, None, :] - cells[None, :, :]) ** 2, axis=-1)
    _, nbr_cells = jax.lax.top_k(-d_cc, _N_NBR_CELLS)

    closest_f = pl.pallas_call(
        _closest_atoms_kernel,
        out_shape=jax.ShapeDtypeStruct((_N_CELLS, _M_NBRS), jnp.float32),
    )(cells, coords.T)
    closest_atoms = closest_f.astype(jnp.int32)

    cand = closest_atoms[nbr_cells].reshape(_N_CELLS, _N_CAND)
    order = jnp.argsort(cand, axis=1)
    sorted_c = jnp.take_along_axis(cand, order, axis=1)
    dup_sorted = jnp.concatenate(
        [
            jnp.zeros((_N_CELLS, 1), dtype=bool),
            sorted_c[:, 1:] == sorted_c[:, :-1],
        ],
        axis=1,
    )
    inv = jnp.argsort(order, axis=1)
    dup = jnp.take_along_axis(dup_sorted, inv, axis=1)

    coords_pad = jnp.pad(coords, ((0, 0), (0, 16 - _NDIM)))
    sc_gather = _make_sc_gather(_N_ATOMS, 16, _N_CELLS * _N_CAND)
    gathered = sc_gather(coords_pad, cand.reshape(-1))
    cand_coords = gathered[:, :_NDIM].reshape(_N_CELLS, _N_CAND, _NDIM)

    npad = _PAD_CAND - _N_CAND
    far = jnp.full((_N_CELLS, npad), 1e15, jnp.float32)
    zpad = jnp.zeros((_N_CELLS, npad), jnp.float32)
    tab = jnp.concatenate(
        [
            cand_coords[:, :, 0], far,
            cand_coords[:, :, 1], far,
            cand_coords[:, :, 2], far,
            cand.astype(jnp.float32), zpad,
            dup.astype(jnp.float32), zpad,
        ],
        axis=1,
    )

    nblocks = _N_ATOMS // _BLOCK_B
    out_f = pl.pallas_call(
        _nbr_list_kernel,
        grid=(nblocks,),
        in_specs=[
            pl.BlockSpec((_NDIM, _N_CELLS), lambda i: (0, 0)),
            pl.BlockSpec((_BLOCK_B, _NDIM), lambda i: (i, 0)),
            pl.BlockSpec((_N_CELLS, 5 * _PAD_CAND), lambda i: (0, 0)),
        ],
        out_specs=pl.BlockSpec((_BLOCK_B, _M_NBRS), lambda i: (i, 0)),
        out_shape=jax.ShapeDtypeStruct((_N_ATOMS, _M_NBRS), jnp.float32),
        compiler_params=pltpu.CompilerParams(
            dimension_semantics=("parallel",)
        ),
    )(cellst, coords, tab)
    return out_f.astype(jnp.int32)
